# Initial kernel scaffold; baseline (speedup 1.0000x reference)
import jax
import jax.numpy as jnp
from jax.experimental import pallas as pl


def kernel(token_ids, weights):
    raise NotImplementedError("write your pallas kernel here")


# speedup vs baseline: 4.8078x; 4.8078x over previous
import functools

import jax
import jax.numpy as jnp
from jax import lax
from jax.experimental import pallas as pl
from jax.experimental.pallas import tpu as pltpu
from jax.experimental.pallas import tpu_sc as plsc

_NUM_CORES = 2
_NUM_SUBCORES = 16
_NW = _NUM_CORES * _NUM_SUBCORES

_LANES = 128
_SUB = 8
_CHUNK = _SUB * _LANES


def _embedding_lookup(idx2d, table, B, D):
    n_idx_rows = B // _LANES
    rows_per_w = B // _NW
    idx_rows_per_w = rows_per_w // _LANES
    n_chunks = idx_rows_per_w // _SUB

    mesh = plsc.VectorSubcoreMesh(
        core_axis_name="c",
        subcore_axis_name="s",
        num_cores=_NUM_CORES,
        num_subcores=_NUM_SUBCORES,
    )

    @functools.partial(
        pl.kernel,
        out_type=jax.ShapeDtypeStruct((B, D), jnp.float32),
        mesh=mesh,
        scratch_types=[
            pltpu.VMEM((_SUB, _LANES), jnp.int32),
            pltpu.VMEM((_CHUNK, D), jnp.float32),
            pltpu.SemaphoreType.DMA,
        ],
        compiler_params=pltpu.CompilerParams(use_tc_tiling_on_sc=False),
    )
    def k(idx_hbm, table_hbm, out_hbm, idx_v, rows_v, sem):
        wid = lax.axis_index("s") * _NUM_CORES + lax.axis_index("c")
        idx_row0 = wid * idx_rows_per_w
        out_row0 = wid * rows_per_w

        @pl.loop(0, n_chunks)
        def chunk_loop(g):
            pltpu.sync_copy(
                idx_hbm.at[pl.ds(idx_row0 + g * _SUB, _SUB), :], idx_v)
            copies = []
            for j in range(_SUB):
                copies.append(pltpu.async_copy(
                    table_hbm.at[idx_v.at[j]],
                    rows_v.at[pl.ds(j * _LANES, _LANES), :],
                    sem))
            for c in copies:
                c.wait()
            pltpu.sync_copy(
                rows_v,
                out_hbm.at[pl.ds(out_row0 + g * _CHUNK, _CHUNK), :])

    return k(idx2d, table)


def kernel(token_ids, weights):
    B0, B1 = token_ids.shape
    B = B0 * B1
    V, D = weights.shape
    idx2d = token_ids.reshape(B // _LANES, _LANES).astype(jnp.int32)
    out = _embedding_lookup(idx2d, weights, B, D)
    return out.reshape(B0, B1, D)


# speedup vs baseline: 5.0476x; 1.0499x over previous
import functools

import jax
import jax.numpy as jnp
from jax import lax
from jax.experimental import pallas as pl
from jax.experimental.pallas import tpu as pltpu
from jax.experimental.pallas import tpu_sc as plsc

_NUM_CORES = 2
_NUM_SUBCORES = 16
_NW = _NUM_CORES * _NUM_SUBCORES

_LANES = 128
_SUB = 4
_CHUNK = _SUB * _LANES
_NBUF = 4


def _embedding_lookup(idx2d, table, B, D):
    rows_per_w = B // _NW
    idx_rows_per_w = rows_per_w // _LANES
    n_chunks = idx_rows_per_w // _SUB

    mesh = plsc.VectorSubcoreMesh(
        core_axis_name="c",
        subcore_axis_name="s",
        num_cores=_NUM_CORES,
        num_subcores=_NUM_SUBCORES,
    )

    @functools.partial(
        pl.kernel,
        out_type=jax.ShapeDtypeStruct((B, D), jnp.float32),
        mesh=mesh,
        scratch_types=[
            pltpu.VMEM((_NBUF * _SUB, _LANES), jnp.int32),
            pltpu.VMEM((_NBUF * _CHUNK, D), jnp.float32),
            [pltpu.SemaphoreType.DMA] * _NBUF,
            [pltpu.SemaphoreType.DMA] * _NBUF,
        ],
        compiler_params=pltpu.CompilerParams(use_tc_tiling_on_sc=False),
    )
    def k(idx_hbm, table_hbm, out_hbm, idx_v, rows_v, sem_g, sem_o):
        wid = lax.axis_index("s") * _NUM_CORES + lax.axis_index("c")
        idx_row0 = wid * idx_rows_per_w
        out_row0 = wid * rows_per_w

        def copy_idx(c, b):
            pltpu.sync_copy(
                idx_hbm.at[pl.ds(idx_row0 + c * _SUB, _SUB), :],
                idx_v.at[pl.ds(b * _SUB, _SUB), :])

        def gathers(b):
            return [
                pltpu.make_async_copy(
                    table_hbm.at[idx_v.at[b * _SUB + j]],
                    rows_v.at[pl.ds((b * _SUB + j) * _LANES, _LANES), :],
                    sem_g[b])
                for j in range(_SUB)
            ]

        def out_copy(c, b):
            return pltpu.make_async_copy(
                rows_v.at[pl.ds(b * _CHUNK, _CHUNK), :],
                out_hbm.at[pl.ds(out_row0 + c * _CHUNK, _CHUNK), :],
                sem_o[b])

        def fire_gathers(b):
            for g in gathers(b):
                g.start()

        def wait_gathers(b):
            for g in gathers(b):
                g.wait()

        for c in (0, 1):
            copy_idx(c, c)
            fire_gathers(c)
        for c in (0, 1):
            copy_idx(c + 2, c + 2)
            fire_gathers(c + 2)
            wait_gathers(c)
            out_copy(c, c).start()

        @pl.loop(2, n_chunks - 2, step=_NBUF)
        def steady(cv):
            for b_off in range(_NBUF):
                c = cv + b_off
                b = (2 + b_off) % _NBUF
                bn = (b + 2) % _NBUF
                out_copy(c - 2, bn).wait()
                copy_idx(c + 2, bn)
                fire_gathers(bn)
                wait_gathers(b)
                out_copy(c, b).start()

        for c in (n_chunks - 2, n_chunks - 1):
            b = c % _NBUF
            out_copy(c - 2, (b + 2) % _NBUF).wait()
            wait_gathers(b)
            out_copy(c, b).start()
        for c in (n_chunks - 2, n_chunks - 1):
            out_copy(c, c % _NBUF).wait()

    return k(idx2d, table)


def kernel(token_ids, weights):
    B0, B1 = token_ids.shape
    B = B0 * B1
    V, D = weights.shape
    idx2d = token_ids.reshape(B // _LANES, _LANES).astype(jnp.int32)
    out = _embedding_lookup(idx2d, weights, B, D)
    return out.reshape(B0, B1, D)
